# Initial kernel scaffold; baseline (speedup 1.0000x reference)
import jax
import jax.numpy as jnp
from jax.experimental import pallas as pl


def kernel(x, q_levels):
    raise NotImplementedError("write your pallas kernel here")


# speedup vs baseline: 2.2687x; 2.2687x over previous
import functools

import jax
import jax.numpy as jnp
from jax import lax
from jax.experimental import pallas as pl
from jax.experimental.pallas import tpu as pltpu
from jax.experimental.pallas import tpu_sc as plsc

H = 32
W = 32
HW = H * W
NQ = 40
C = HW * NQ
LANES = 16
NVEC = C // LANES
NCORES = 2
NSUB = 16
NW = NCORES * NSUB
ROWS_PER_W = HW // NW


def _sc_labels(a_rep, b_tile, tsplat):
    mesh = plsc.VectorSubcoreMesh(core_axis_name="c", subcore_axis_name="s")

    @functools.partial(
        pl.kernel,
        mesh=mesh,
        out_type=jax.ShapeDtypeStruct((HW, C), jnp.float32),
        scratch_types=[
            pltpu.VMEM((C,), jnp.float32),
            pltpu.VMEM((C,), jnp.float32),
            pltpu.VMEM((C,), jnp.float32),
            pltpu.VMEM((ROWS_PER_W * LANES,), jnp.float32),
            pltpu.SemaphoreType.DMA,
            pltpu.SemaphoreType.DMA,
        ],
    )
    def sc_kernel(a_hbm, b_hbm, tsplat_hbm, out_hbm,
                  ratio_v, buf0, buf1, tsplat_v, sem0, sem1):
        cid = lax.axis_index("c")
        sid = lax.axis_index("s")
        wid = cid * NSUB + sid
        base = wid * ROWS_PER_W

        pltpu.sync_copy(a_hbm, buf0)
        pltpu.sync_copy(b_hbm, buf1)
        pltpu.sync_copy(
            tsplat_hbm.at[pl.ds(base * LANES, ROWS_PER_W * LANES)], tsplat_v)

        def rdiv(j, carry):
            s = pl.ds(j * LANES, LANES)
            ratio_v[s] = buf0[s] / buf1[s]
            return carry

        lax.fori_loop(0, NVEC, rdiv, 0)

        bufs = (buf0, buf1)
        sems = (sem0, sem1)

        def row_pair(g, carry):
            for b in range(2):
                rl = g * 2 + b
                r = base + rl
                tvec = tsplat_v[pl.ds(rl * LANES, LANES)]

                @pl.when(g > 0)
                def _wait():
                    pltpu.make_async_copy(
                        bufs[b], out_hbm.at[base], sems[b]).wait()

                def compute(j, inner_carry):
                    s = pl.ds(j * LANES, LANES)
                    bufs[b][s] = jnp.where(ratio_v[s] >= tvec, 1.0, 0.0)
                    return inner_carry

                lax.fori_loop(0, NVEC, compute, 0)
                pltpu.async_copy(bufs[b], out_hbm.at[r], sems[b])
            return carry

        lax.fori_loop(0, ROWS_PER_W // 2, row_pair, 0)

        for b in range(2):
            pltpu.make_async_copy(bufs[b], out_hbm.at[base], sems[b]).wait()

    return sc_kernel(a_rep, b_tile, tsplat)


def kernel(x, q_levels):
    flat = x.reshape(HW)
    a_rep = jnp.repeat(flat, NQ)
    b_tile = jnp.tile(q_levels, HW)
    tsplat = jnp.repeat(flat, LANES)
    out = _sc_labels(a_rep, b_tile, tsplat)
    return out.reshape(H, W, HW * NQ)


# speedup vs baseline: 8.1660x; 3.5994x over previous
import functools

import jax
import jax.numpy as jnp
from jax import lax
from jax.experimental import pallas as pl
from jax.experimental.pallas import tpu as pltpu
from jax.experimental.pallas import tpu_sc as plsc

H = 32
W = 32
HW = H * W
NQ = 40
C = HW * NQ
LANES = 16
NVEC = C // LANES
NCORES = 2
NSUB = 16
NW = NCORES * NSUB
ROWS_PER_W = HW // NW
UNROLL = 8


def _sc_labels(a_rep, b_tile, tsplat):
    mesh = plsc.VectorSubcoreMesh(core_axis_name="c", subcore_axis_name="s")

    @functools.partial(
        pl.kernel,
        mesh=mesh,
        out_type=jax.ShapeDtypeStruct((HW, C), jnp.float32),
        scratch_types=[
            pltpu.VMEM((C,), jnp.float32),
            pltpu.VMEM((C,), jnp.float32),
            pltpu.VMEM((C,), jnp.float32),
            pltpu.VMEM((ROWS_PER_W * LANES,), jnp.float32),
            pltpu.SemaphoreType.DMA,
            pltpu.SemaphoreType.DMA,
        ],
    )
    def sc_kernel(a_hbm, b_hbm, tsplat_hbm, out_hbm,
                  ratio_v, buf0, buf1, tsplat_v, sem0, sem1):
        cid = lax.axis_index("c")
        sid = lax.axis_index("s")
        wid = cid * NSUB + sid
        base = wid * ROWS_PER_W

        pltpu.sync_copy(a_hbm, buf0)
        pltpu.sync_copy(b_hbm, buf1)
        pltpu.sync_copy(
            tsplat_hbm.at[pl.ds(base * LANES, ROWS_PER_W * LANES)], tsplat_v)

        def rdiv(j, carry):
            for u in range(UNROLL):
                s = pl.ds((j * UNROLL + u) * LANES, LANES)
                ratio_v[s] = buf0[s] / buf1[s]
            return carry

        lax.fori_loop(0, NVEC // UNROLL, rdiv, 0)

        bufs = (buf0, buf1)
        sems = (sem0, sem1)

        def row_pair(g, carry):
            for b in range(2):
                rl = g * 2 + b
                r = base + rl
                tvec = tsplat_v[pl.ds(rl * LANES, LANES)]

                @pl.when(g > 0)
                def _wait():
                    pltpu.make_async_copy(
                        bufs[b], out_hbm.at[base], sems[b]).wait()

                def compute(j, inner_carry):
                    for u in range(UNROLL):
                        s = pl.ds((j * UNROLL + u) * LANES, LANES)
                        bufs[b][s] = jnp.where(ratio_v[s] >= tvec, 1.0, 0.0)
                    return inner_carry

                lax.fori_loop(0, NVEC // UNROLL, compute, 0)
                pltpu.async_copy(bufs[b], out_hbm.at[r], sems[b])
            return carry

        lax.fori_loop(0, ROWS_PER_W // 2, row_pair, 0)

        for b in range(2):
            pltpu.make_async_copy(bufs[b], out_hbm.at[base], sems[b]).wait()

    return sc_kernel(a_rep, b_tile, tsplat)


def kernel(x, q_levels):
    flat = x.reshape(HW)
    a_rep = jnp.repeat(flat, NQ)
    b_tile = jnp.tile(q_levels, HW)
    tsplat = jnp.repeat(flat, LANES)
    out = _sc_labels(a_rep, b_tile, tsplat)
    return out.reshape(H, W, HW * NQ)


# speedup vs baseline: 8.4284x; 1.0321x over previous
import functools

import jax
import jax.numpy as jnp
from jax import lax
from jax.experimental import pallas as pl
from jax.experimental.pallas import tpu as pltpu
from jax.experimental.pallas import tpu_sc as plsc

H = 32
W = 32
HW = H * W
NQ = 40
C = HW * NQ
LANES = 16
NVEC = C // LANES
NCORES = 2
NSUB = 16
NW = NCORES * NSUB
ROWS_PER_W = HW // NW
UNROLL = 8


def _sc_labels(a_rep, b_tile, tsplat, nrows):
    rpw = nrows // NW
    mesh = plsc.VectorSubcoreMesh(core_axis_name="c", subcore_axis_name="s")

    @functools.partial(
        pl.kernel,
        mesh=mesh,
        out_type=jax.ShapeDtypeStruct((nrows, C), jnp.float32),
        scratch_types=[
            pltpu.VMEM((C,), jnp.float32),
            pltpu.VMEM((C,), jnp.float32),
            pltpu.VMEM((C,), jnp.float32),
            pltpu.VMEM((rpw * LANES,), jnp.float32),
            pltpu.SemaphoreType.DMA,
            pltpu.SemaphoreType.DMA,
        ],
    )
    def sc_kernel(a_hbm, b_hbm, tsplat_hbm, out_hbm,
                  ratio_v, buf0, buf1, tsplat_v, sem0, sem1):
        cid = lax.axis_index("c")
        sid = lax.axis_index("s")
        wid = cid * NSUB + sid
        base = wid * rpw

        pltpu.sync_copy(a_hbm, buf0)
        pltpu.sync_copy(b_hbm, buf1)
        pltpu.sync_copy(
            tsplat_hbm.at[pl.ds(base * LANES, rpw * LANES)], tsplat_v)

        def rdiv(j, carry):
            for u in range(UNROLL):
                s = pl.ds((j * UNROLL + u) * LANES, LANES)
                ratio_v[s] = buf0[s] / buf1[s]
            return carry

        lax.fori_loop(0, NVEC // UNROLL, rdiv, 0)

        bufs = (buf0, buf1)
        sems = (sem0, sem1)

        def row_pair(g, carry):
            for b in range(2):
                rl = g * 2 + b
                r = base + rl
                tvec = tsplat_v[pl.ds(rl * LANES, LANES)]

                @pl.when(g > 0)
                def _wait():
                    pltpu.make_async_copy(
                        bufs[b], out_hbm.at[base], sems[b]).wait()

                def compute(j, inner_carry):
                    for u in range(UNROLL):
                        s = pl.ds((j * UNROLL + u) * LANES, LANES)
                        bufs[b][s] = jnp.where(ratio_v[s] >= tvec, 1.0, 0.0)
                    return inner_carry

                lax.fori_loop(0, NVEC // UNROLL, compute, 0)
                pltpu.async_copy(bufs[b], out_hbm.at[r], sems[b])
            return carry

        lax.fori_loop(0, rpw // 2, row_pair, 0)

        for b in range(2):
            pltpu.make_async_copy(bufs[b], out_hbm.at[base], sems[b]).wait()

    return sc_kernel(a_rep, b_tile, tsplat)


def _tc_body(ratio_ref, rowm_ref, out_ref):
    t = rowm_ref[:, 0:1]
    out_ref[...] = jnp.where(ratio_ref[...] >= t, 1.0, 0.0)


def _tc_ratio_body(a_ref, b_ref, out_ref):
    out_ref[...] = a_ref[...] / b_ref[...]


def _tc_labels(a_rep, b_tile, rowm, nrows):
    BR = 32
    ratio = pl.pallas_call(
        _tc_ratio_body,
        out_shape=jax.ShapeDtypeStruct((1, C), jnp.float32),
    )(a_rep[None, :], b_tile[None, :])
    return pl.pallas_call(
        _tc_body,
        grid=(nrows // BR,),
        in_specs=[
            pl.BlockSpec((1, C), lambda i: (0, 0)),
            pl.BlockSpec((BR, 128), lambda i: (i, 0)),
        ],
        out_specs=pl.BlockSpec((BR, C), lambda i: (i, 0)),
        out_shape=jax.ShapeDtypeStruct((nrows, C), jnp.float32),
    )(ratio, rowm)


def kernel(x, q_levels):
    flat = x.reshape(HW)
    a_rep = jnp.repeat(flat, NQ)
    b_tile = jnp.tile(q_levels, HW)
    split = 512
    tsplat = jnp.repeat(flat[split:], LANES)
    sc_out = _sc_labels(a_rep, b_tile, tsplat, HW - split)
    rowm = jnp.broadcast_to(flat[:split, None], (split, 128))
    tc_out = _tc_labels(a_rep, b_tile, rowm, split)
    return (tc_out, sc_out)


# speedup vs baseline: 12.5790x; 1.4925x over previous
import functools

import jax
import jax.numpy as jnp
from jax import lax
from jax.experimental import pallas as pl
from jax.experimental.pallas import tpu as pltpu
from jax.experimental.pallas import tpu_sc as plsc

H = 32
W = 32
HW = H * W
NQ = 40
C = HW * NQ
LANES = 16
NVEC = C // LANES
NCORES = 2
NSUB = 16
NW = NCORES * NSUB
ROWS_PER_W = HW // NW
UNROLL = 8


def _sc_labels(a_rep, b_tile, tsplat, nrows):
    rpw = nrows // NW
    mesh = plsc.VectorSubcoreMesh(core_axis_name="c", subcore_axis_name="s")

    @functools.partial(
        pl.kernel,
        mesh=mesh,
        out_type=jax.ShapeDtypeStruct((nrows, C), jnp.float32),
        scratch_types=[
            pltpu.VMEM((C,), jnp.float32),
            pltpu.VMEM((C,), jnp.float32),
            pltpu.VMEM((C,), jnp.float32),
            pltpu.VMEM((rpw * LANES,), jnp.float32),
            pltpu.SemaphoreType.DMA,
            pltpu.SemaphoreType.DMA,
        ],
    )
    def sc_kernel(a_hbm, b_hbm, tsplat_hbm, out_hbm,
                  ratio_v, buf0, buf1, tsplat_v, sem0, sem1):
        cid = lax.axis_index("c")
        sid = lax.axis_index("s")
        wid = cid * NSUB + sid
        base = wid * rpw

        pltpu.sync_copy(a_hbm, buf0)
        pltpu.sync_copy(b_hbm, buf1)
        pltpu.sync_copy(
            tsplat_hbm.at[pl.ds(base * LANES, rpw * LANES)], tsplat_v)

        def rdiv(j, carry):
            for u in range(UNROLL):
                s = pl.ds((j * UNROLL + u) * LANES, LANES)
                ratio_v[s] = buf0[s] / buf1[s]
            return carry

        lax.fori_loop(0, NVEC // UNROLL, rdiv, 0)

        bufs = (buf0, buf1)
        sems = (sem0, sem1)

        def row_pair(g, carry):
            for b in range(2):
                rl = g * 2 + b
                r = base + rl
                tvec = tsplat_v[pl.ds(rl * LANES, LANES)]

                @pl.when(g > 0)
                def _wait():
                    pltpu.make_async_copy(
                        bufs[b], out_hbm.at[base], sems[b]).wait()

                def compute(j, inner_carry):
                    for u in range(UNROLL):
                        s = pl.ds((j * UNROLL + u) * LANES, LANES)
                        bufs[b][s] = jnp.where(ratio_v[s] >= tvec, 1.0, 0.0)
                    return inner_carry

                lax.fori_loop(0, NVEC // UNROLL, compute, 0)
                pltpu.async_copy(bufs[b], out_hbm.at[r], sems[b])
            return carry

        lax.fori_loop(0, rpw // 2, row_pair, 0)

        for b in range(2):
            pltpu.make_async_copy(bufs[b], out_hbm.at[base], sems[b]).wait()

    return sc_kernel(a_rep, b_tile, tsplat)


def _tc_body(ratio_ref, rowm_ref, out_ref):
    t = rowm_ref[:, 0:1]
    out_ref[...] = jnp.where(ratio_ref[...] >= t, 1.0, 0.0)


def _tc_ratio_body(a_ref, b_ref, out_ref):
    out_ref[...] = a_ref[...] / b_ref[...]


def _tc_labels(a_rep, b_tile, rowm, nrows):
    BR = 32
    ratio = pl.pallas_call(
        _tc_ratio_body,
        out_shape=jax.ShapeDtypeStruct((1, C), jnp.float32),
    )(a_rep[None, :], b_tile[None, :])
    return pl.pallas_call(
        _tc_body,
        grid=(nrows // BR,),
        in_specs=[
            pl.BlockSpec((1, C), lambda i: (0, 0)),
            pl.BlockSpec((BR, 128), lambda i: (i, 0)),
        ],
        out_specs=pl.BlockSpec((BR, C), lambda i: (i, 0)),
        out_shape=jax.ShapeDtypeStruct((nrows, C), jnp.float32),
    )(ratio, rowm)


def kernel(x, q_levels):
    flat = x.reshape(HW)
    a_rep = jnp.repeat(flat, NQ)
    b_tile = jnp.tile(q_levels, HW)
    split = HW
    rowm = jnp.broadcast_to(flat[:split, None], (split, 128))
    tc_out = _tc_labels(a_rep, b_tile, rowm, split)
    return (tc_out,)
